# baseline (speedup 1.0000x reference)
import functools

import jax
import jax.numpy as jnp
from jax import lax
from jax.experimental import pallas as pl
from jax.experimental.pallas import tpu as pltpu
from jax.experimental.pallas import tpu_sc as plsc

_NC = 2
_NS = 16
_NW = _NC * _NS
_CHUNK = 128

_BM = 1024
_CAT = 128


@functools.partial(jax.jit, static_argnames=("b", "d"))
def _gather_uid(uid_table, idx2d, b, d):
    bpw = b // _NW
    nch = bpw // _CHUNK
    mesh = plsc.VectorSubcoreMesh(core_axis_name="c", subcore_axis_name="s")

    @functools.partial(
        pl.kernel,
        mesh=mesh,
        out_type=jax.ShapeDtypeStruct((b, d), jnp.float32),
        scratch_types=[
            pltpu.VMEM((nch, _CHUNK), jnp.int32),
            pltpu.VMEM((bpw, d), jnp.float32),
            pltpu.SemaphoreType.DMA,
        ],
        compiler_params=pltpu.CompilerParams(use_tc_tiling_on_sc=False),
    )
    def gather(table_hbm, idx_hbm, out_hbm, idx_v, rows_v, sem):
        wid = lax.axis_index("s") * _NC + lax.axis_index("c")
        pltpu.sync_copy(idx_hbm.at[pl.ds(wid * nch, nch)], idx_v)
        copies = []
        for j in range(nch):
            copies.append(pltpu.async_copy(
                table_hbm.at[idx_v.at[j]],
                rows_v.at[pl.ds(j * _CHUNK, _CHUNK)], sem))
        for c in copies:
            c.wait()
        pltpu.sync_copy(rows_v, out_hbm.at[pl.ds(wid * bpw, bpw)])

    return gather(uid_table, idx2d)


def _mlp_block(gender_ref, age_ref, uid_ref, w1u_ref, ct_ref, w1ga_ref,
               b1_ref, w2_ref, b2_ref, out_ref):
    g = gender_ref[0, 0, :]
    a = age_ref[0, 0, :]
    iota = lax.broadcasted_iota(jnp.int32, (_BM, _CAT), 1)
    onehot = jnp.logical_or(iota == g[:, None],
                            iota == a[:, None] + 3).astype(jnp.float32)
    ga = jnp.dot(ct_ref[...], w1ga_ref[...], preferred_element_type=jnp.float32)
    h = jnp.dot(uid_ref[...], w1u_ref[...], preferred_element_type=jnp.float32)
    h = h + jnp.dot(onehot, ga, preferred_element_type=jnp.float32) + b1_ref[...]
    h = jnp.where(h >= 0, h, 0.01 * h)
    out_ref[...] = (jnp.dot(h, w2_ref[...], preferred_element_type=jnp.float32)
                    + b2_ref[...])


def kernel(uid, gender, age, uid_table, gender_table, age_table, W1, b1, W2, b2):
    b = uid.shape[0]
    v_uid, user_dim = uid_table.shape
    g_dim = gender_table.shape[1]
    a_dim = age_table.shape[1]
    model_dim = W2.shape[1]

    idx2d = uid.astype(jnp.int32).reshape(b // _CHUNK, _CHUNK)
    uid_emb = _gather_uid(uid_table, idx2d, b, user_dim)

    ct = jnp.zeros((_CAT, g_dim + a_dim), jnp.float32)
    ct = ct.at[0:gender_table.shape[0], 0:g_dim].set(gender_table)
    ct = ct.at[3:3 + age_table.shape[0], g_dim:].set(age_table)

    nb = b // _BM
    out = pl.pallas_call(
        _mlp_block,
        grid=(nb,),
        in_specs=[
            pl.BlockSpec((1, 1, _BM), lambda i: (i, 0, 0)),
            pl.BlockSpec((1, 1, _BM), lambda i: (i, 0, 0)),
            pl.BlockSpec((_BM, user_dim), lambda i: (i, 0)),
            pl.BlockSpec((user_dim, model_dim), lambda i: (0, 0)),
            pl.BlockSpec((_CAT, g_dim + a_dim), lambda i: (0, 0)),
            pl.BlockSpec((g_dim + a_dim, model_dim), lambda i: (0, 0)),
            pl.BlockSpec((1, model_dim), lambda i: (0, 0)),
            pl.BlockSpec((model_dim, model_dim), lambda i: (0, 0)),
            pl.BlockSpec((1, model_dim), lambda i: (0, 0)),
        ],
        out_specs=pl.BlockSpec((_BM, model_dim), lambda i: (i, 0)),
        out_shape=jax.ShapeDtypeStruct((b, model_dim), jnp.float32),
        compiler_params=pltpu.CompilerParams(
            dimension_semantics=("arbitrary",)),
    )(gender.reshape(nb, 1, _BM), age.reshape(nb, 1, _BM), uid_emb,
      W1[:user_dim], ct, W1[user_dim:], b1.reshape(1, model_dim), W2,
      b2.reshape(1, model_dim))
    return out[:, None, :]
